# baseline (speedup 1.0000x reference)
import functools

import jax
import jax.numpy as jnp
from jax import lax
from jax.experimental import pallas as pl
from jax.experimental.pallas import tpu as pltpu
from jax.experimental.pallas import tpu_sc as plsc

_SEQ = 8192
_DIM = 1024
_BATCH = 4
_NW = 32
_ROWS_PER_W = _SEQ // _NW
_CH_ROWS = 16
_CH = _CH_ROWS * _DIM
_NCH = _ROWS_PER_W // _CH_ROWS


def _sc_body(x_hbm, t_hbm, o_hbm, buf, tbuf):
    wid = lax.axis_index("s") * 2 + lax.axis_index("c")
    s_base = wid * _ROWS_PER_W

    def chunk_body(i, carry):
        s_off = s_base + i * _CH_ROWS
        toff = s_off * _DIM
        pltpu.sync_copy(t_hbm.at[pl.ds(toff, _CH)], tbuf)
        for b in range(_BATCH):
            off = (b * _SEQ + s_off) * _DIM
            pltpu.sync_copy(x_hbm.at[pl.ds(off, _CH)], buf)

            @plsc.parallel_loop(0, _CH, step=16, unroll=8)
            def add_body(j):
                plsc.addupdate(buf.at[pl.ds(j, 16)], tbuf[pl.ds(j, 16)])

            pltpu.sync_copy(buf, o_hbm.at[pl.ds(off, _CH)])
        return carry

    lax.fori_loop(0, _NCH, chunk_body, 0)


@functools.partial(
    pl.kernel,
    out_type=jax.ShapeDtypeStruct((_BATCH * _SEQ * _DIM,), jnp.float32),
    mesh=plsc.VectorSubcoreMesh(core_axis_name="c", subcore_axis_name="s"),
    scratch_types=[
        pltpu.VMEM((_CH,), jnp.float32),
        pltpu.VMEM((_CH,), jnp.float32),
    ],
)
def _sc_add(x_hbm, t_hbm, o_hbm, buf, tbuf):
    _sc_body(x_hbm, t_hbm, o_hbm, buf, tbuf)


def kernel(inputs, pos_table):
    batch, seq, dim = inputs.shape
    out = _sc_add(inputs.reshape(-1), pos_table.reshape(-1))
    return out.reshape(batch, seq, dim)
